# baseline (speedup 1.0000x reference)
import functools
import math

import numpy as np
import jax
import jax.numpy as jnp
from jax import lax
from jax.experimental import pallas as pl
from jax.experimental.pallas import tpu as pltpu
from jax.experimental.pallas import tpu_sc as plsc

VOCAB = 100000
SEQ = 2048
D = 1024
B = 4
NC = 2
NS = 16
NW = NC * NS
ROWS_PER_W = (B * SEQ) // NW
C = 16
NCHUNK = ROWS_PER_W // C
SEQ_PER_W = SEQ // ROWS_PER_W
SCALE = math.sqrt(D)


def _pe_flat() -> np.ndarray:
    pos = np.arange(SEQ, dtype=np.float32)[:, None]
    div = np.exp(np.arange(0, D, 2, dtype=np.float32) * (-math.log(10000.0) / D))
    pe = np.zeros((SEQ, D), np.float32)
    pe[:, 0::2] = np.sin(pos * div)
    pe[:, 1::2] = np.cos(pos * div)
    return pe.reshape(-1)


_PE = _pe_flat()


def _embed_kernel(tok_hbm, w_hbm, pe_hbm, out_hbm, idx_v, rows_v, peb_v, gsem):
    wid = lax.axis_index("s") * NC + lax.axis_index("c")
    row_base = wid * ROWS_PER_W
    pos_base = lax.rem(wid, SEQ // ROWS_PER_W) * ROWS_PER_W

    pltpu.sync_copy(tok_hbm.at[wid], idx_v)

    for c in range(NCHUNK):
        pltpu.async_copy(w_hbm.at[idx_v.at[c]], rows_v, gsem).wait()
        pe_off = (pos_base + c * C) * D
        pltpu.sync_copy(pe_hbm.at[pl.ds(pe_off, C * D)], peb_v)
        for r in range(C):
            def body(j, _, r=r):
                off = r * D + j * 16
                v = rows_v[r, pl.ds(j * 16, 16)]
                peb_v[pl.ds(off, 16)] = peb_v[pl.ds(off, 16)] + v * SCALE
                return _
            lax.fori_loop(0, D // 16, body, None)
        out_off = (row_base + c * C) * D
        pltpu.sync_copy(peb_v, out_hbm.at[pl.ds(out_off, C * D)])


def kernel(tokens, W):
    tok = tokens.astype(jnp.int32).reshape(NW, NCHUNK, C)
    pe = jnp.asarray(_PE)
    mesh = plsc.VectorSubcoreMesh(
        core_axis_name="c", subcore_axis_name="s", num_cores=NC, num_subcores=NS
    )
    run = pl.kernel(
        _embed_kernel,
        out_type=jax.ShapeDtypeStruct((B * SEQ * D,), jnp.float32),
        mesh=mesh,
        scratch_types=[
            pltpu.VMEM((NCHUNK, C), jnp.int32),
            pltpu.VMEM((C, D), jnp.float32),
            pltpu.VMEM((C * D,), jnp.float32),
            pltpu.SemaphoreType.DMA,
        ],
    )
    out = run(tok, W, pe)
    return out.reshape(B, SEQ, D)


# speedup vs baseline: 2.2648x; 2.2648x over previous
import math

import numpy as np
import jax
import jax.numpy as jnp
from jax import lax
from jax.experimental import pallas as pl
from jax.experimental.pallas import tpu as pltpu
from jax.experimental.pallas import tpu_sc as plsc

VOCAB = 100000
SEQ = 2048
D = 1024
B = 4
NC = 2
NS = 16
NW = NC * NS
ROWS_PER_W = (B * SEQ) // NW
C = 16
NCHUNK = ROWS_PER_W // C
NB_R = 3
NB_P = 4
PE_AHEAD = 3
SCALE = math.sqrt(D)


def _pe_flat() -> np.ndarray:
    pos = np.arange(SEQ, dtype=np.float32)[:, None]
    div = np.exp(np.arange(0, D, 2, dtype=np.float32) * (-math.log(10000.0) / D))
    pe = np.zeros((SEQ, D), np.float32)
    pe[:, 0::2] = np.sin(pos * div)
    pe[:, 1::2] = np.cos(pos * div)
    return pe.reshape(-1)


_PE = _pe_flat()


def _embed_kernel(tok_hbm, w_hbm, pe_hbm, out_hbm, idx_v, rows, peb, gsems, psems, osems):
    wid = lax.axis_index("s") * NC + lax.axis_index("c")
    row_base = wid * ROWS_PER_W
    pos_base = lax.rem(wid, SEQ // ROWS_PER_W) * ROWS_PER_W

    pltpu.sync_copy(tok_hbm.at[wid], idx_v)

    gh, ph, oh = {}, {}, {}

    def start_gather(c):
        gh[c] = pltpu.async_copy(w_hbm.at[idx_v.at[c]], rows[c % NB_R], gsems[c % NB_R])

    def start_pe(c):
        off = (pos_base + c * C) * D
        ph[c] = pltpu.async_copy(pe_hbm.at[pl.ds(off, C * D)], peb[c % NB_P], psems[c % NB_P])

    def start_out(c):
        off = (row_base + c * C) * D
        oh[c] = pltpu.async_copy(peb[c % NB_P], out_hbm.at[pl.ds(off, C * D)], osems[c % NB_P])

    def compute(rv, pv):
        def rbody(r, carry):
            @plsc.parallel_loop(0, D // 16, unroll=8)
            def jbody(j):
                off = j * 16
                v = rv[r, pl.ds(off, 16)]
                plsc.addupdate(pv.at[pl.ds(r * D + off, 16)], v * SCALE)
            return carry
        lax.fori_loop(0, C, rbody, None)

    for c in range(NB_R):
        start_gather(c)
    for c in range(PE_AHEAD):
        start_pe(c)

    waited_outs = set()
    for c in range(NCHUNK):
        gh[c].wait()
        ph[c].wait()
        compute(rows[c % NB_R], peb[c % NB_P])
        start_out(c)
        if c + NB_R < NCHUNK:
            start_gather(c + NB_R)
        n = c + PE_AHEAD
        if n < NCHUNK:
            prev = n - NB_P
            if prev >= 0:
                oh[prev].wait()
                waited_outs.add(prev)
            start_pe(n)
    for c in range(NCHUNK):
        if c not in waited_outs:
            oh[c].wait()


def kernel(tokens, W):
    tok = tokens.astype(jnp.int32).reshape(NW, NCHUNK, C)
    pe = jnp.asarray(_PE)
    mesh = plsc.VectorSubcoreMesh(
        core_axis_name="c", subcore_axis_name="s", num_cores=NC, num_subcores=NS
    )
    run = pl.kernel(
        _embed_kernel,
        out_type=jax.ShapeDtypeStruct((B * SEQ * D,), jnp.float32),
        mesh=mesh,
        scratch_types=[
            pltpu.VMEM((NCHUNK, C), jnp.int32),
            [pltpu.VMEM((C, D), jnp.float32) for _ in range(NB_R)],
            [pltpu.VMEM((C * D,), jnp.float32) for _ in range(NB_P)],
            [pltpu.SemaphoreType.DMA for _ in range(NB_R)],
            [pltpu.SemaphoreType.DMA for _ in range(NB_P)],
            [pltpu.SemaphoreType.DMA for _ in range(NB_P)],
        ],
    )
    out = run(tok, W, pe)
    return out.reshape(B, SEQ, D)


# speedup vs baseline: 3.5240x; 1.5560x over previous
import math

import numpy as np
import jax
import jax.numpy as jnp
from jax import lax
from jax.experimental import pallas as pl
from jax.experimental.pallas import tpu as pltpu
from jax.experimental.pallas import tpu_sc as plsc

VOCAB = 100000
SEQ = 2048
D = 1024
B = 4
NC = 2
NS = 16
NW = NC * NS
ROWS_PER_W = (B * SEQ) // NW
W_PER_SEQ = SEQ // ROWS_PER_W
C = 16
NCHUNK = ROWS_PER_W // C
NB_R = 3
NB_P = 4
PE_AHEAD = 3
SCALE = math.sqrt(D)


def _pe_table() -> np.ndarray:
    pos = np.arange(SEQ, dtype=np.float32)[:, None]
    div = np.exp(np.arange(0, D, 2, dtype=np.float32) * (-math.log(10000.0) / D))
    pe = np.zeros((SEQ, D), np.float32)
    pe[:, 0::2] = np.sin(pos * div)
    pe[:, 1::2] = np.cos(pos * div)
    return pe


_PE = _pe_table()
_PE_DEV = []


def _pe_dev():
    if not _PE_DEV:
        _PE_DEV.append(jax.device_put(jnp.asarray(_PE)))
    return _PE_DEV[0]


def _embed_kernel(tok_hbm, w_hbm, pe_hbm, out_hbm, idx_v, rows, peb, gsems, psems, osems):
    wid = lax.axis_index("s") * NC + lax.axis_index("c")
    seq = lax.div(wid, W_PER_SEQ)
    pos_base = lax.rem(wid, W_PER_SEQ) * ROWS_PER_W

    pltpu.sync_copy(tok_hbm.at[seq, pl.ds(pos_base, ROWS_PER_W)], idx_v)

    gh, ph, oh = {}, {}, {}

    def start_gather(c):
        idx = idx_v.at[pl.ds(c * C, C)]
        gh[c] = pltpu.async_copy(w_hbm.at[idx], rows[c % NB_R], gsems[c % NB_R])

    def start_pe(c):
        src = pe_hbm.at[pl.ds(pos_base + c * C, C)]
        ph[c] = pltpu.async_copy(src, peb[c % NB_P], psems[c % NB_P])

    def start_out(c):
        dst = out_hbm.at[seq, pl.ds(pos_base + c * C, C)]
        oh[c] = pltpu.async_copy(peb[c % NB_P], dst, osems[c % NB_P])

    def compute(rv, pv):
        def rbody(r, carry):
            @plsc.parallel_loop(0, D // 16, unroll=8)
            def jbody(j):
                off = j * 16
                v = rv[r, pl.ds(off, 16)]
                plsc.addupdate(pv.at[r, pl.ds(off, 16)], v * SCALE)
            return carry
        lax.fori_loop(0, C, rbody, None)

    for c in range(NB_R):
        start_gather(c)
    for c in range(PE_AHEAD):
        start_pe(c)

    waited_outs = set()
    for c in range(NCHUNK):
        gh[c].wait()
        ph[c].wait()
        compute(rows[c % NB_R], peb[c % NB_P])
        start_out(c)
        if c + NB_R < NCHUNK:
            start_gather(c + NB_R)
        n = c + PE_AHEAD
        if n < NCHUNK:
            prev = n - NB_P
            if prev >= 0:
                oh[prev].wait()
                waited_outs.add(prev)
            start_pe(n)
    for c in range(NCHUNK):
        if c not in waited_outs:
            oh[c].wait()


def kernel(tokens, W):
    mesh = plsc.VectorSubcoreMesh(
        core_axis_name="c", subcore_axis_name="s", num_cores=NC, num_subcores=NS
    )
    run = pl.kernel(
        _embed_kernel,
        out_type=jax.ShapeDtypeStruct((B, SEQ, D), jnp.float32),
        mesh=mesh,
        scratch_types=[
            pltpu.VMEM((ROWS_PER_W,), jnp.int32),
            [pltpu.VMEM((C, D), jnp.float32) for _ in range(NB_R)],
            [pltpu.VMEM((C, D), jnp.float32) for _ in range(NB_P)],
            [pltpu.SemaphoreType.DMA for _ in range(NB_R)],
            [pltpu.SemaphoreType.DMA for _ in range(NB_P)],
            [pltpu.SemaphoreType.DMA for _ in range(NB_P)],
        ],
    )
    return run(tokens.astype(jnp.int32), W, _pe_dev())


# speedup vs baseline: 4.1053x; 1.1649x over previous
import math

import ml_dtypes
import numpy as np
import jax
import jax.numpy as jnp
from jax import lax
from jax.experimental import pallas as pl
from jax.experimental.pallas import tpu as pltpu
from jax.experimental.pallas import tpu_sc as plsc

VOCAB = 100000
SEQ = 2048
D = 1024
B = 4
NC = 2
NS = 16
NW = NC * NS
ROWS_PER_W = (B * SEQ) // NW
W_PER_SEQ = SEQ // ROWS_PER_W
C = 16
NCHUNK = ROWS_PER_W // C
NB = 3
SCALE = math.sqrt(D)


def _pe_table() -> np.ndarray:
    pos = np.arange(SEQ, dtype=np.float32)[:, None]
    div = np.exp(np.arange(0, D, 2, dtype=np.float32) * (-math.log(10000.0) / D))
    pe = np.zeros((SEQ, D), np.float32)
    pe[:, 0::2] = np.sin(pos * div)
    pe[:, 1::2] = np.cos(pos * div)
    pe = pe.reshape(SEQ * D).astype(ml_dtypes.bfloat16)
    bits = pe.view(np.uint16).reshape(-1, 2, 16).astype(np.uint32)
    words = bits[:, 0, :] | (bits[:, 1, :] << 16)
    return words.reshape(SEQ * D // 2).view(np.int32)


_PE = _pe_table()


def _embed_kernel(tok_hbm, w_hbm, pe_hbm, out_hbm, idx_v, rows, peb, outb, gsems, psems, osems):
    wid = lax.axis_index("s") * NC + lax.axis_index("c")
    seq = lax.div(wid, W_PER_SEQ)
    pos_base = lax.rem(wid, W_PER_SEQ) * ROWS_PER_W

    pltpu.sync_copy(tok_hbm.at[seq, pl.ds(pos_base, ROWS_PER_W)], idx_v)

    gh, ph, oh = {}, {}, {}

    def start_gather(c):
        idx = idx_v.at[pl.ds(c * C, C)]
        gh[c] = pltpu.async_copy(w_hbm.at[idx], rows[c % NB], gsems[c % NB])

    def start_pe(c):
        src = pe_hbm.at[pl.ds((pos_base + c * C) * (D // 2), C * D // 2)]
        ph[c] = pltpu.async_copy(src, peb[c % NB], psems[c % NB])

    def start_out(c):
        dst = out_hbm.at[seq, pl.ds(pos_base + c * C, C)]
        oh[c] = pltpu.async_copy(outb[c % NB], dst, osems[c % NB])

    def compute(rv, pv, ov):
        def rbody(r, carry):
            @plsc.parallel_loop(0, D // 32, unroll=4)
            def jbody(k):
                off = k * 32
                w = pv[pl.ds((r * D + off) // 2, 16)]
                a = lax.bitcast_convert_type(lax.shift_left(w, 16), jnp.float32)
                b = lax.bitcast_convert_type(
                    lax.bitwise_and(w, jnp.int32(-65536)), jnp.float32)
                v0 = rv[r, pl.ds(off, 16)]
                v1 = rv[r, pl.ds(off + 16, 16)]
                ov[r, pl.ds(off, 16)] = a + v0 * SCALE
                ov[r, pl.ds(off + 16, 16)] = b + v1 * SCALE
            return carry
        lax.fori_loop(0, C, rbody, None)

    for c in range(NB):
        start_gather(c)
        start_pe(c)

    for c in range(NCHUNK):
        if c >= NB:
            oh[c - NB].wait()
        gh[c].wait()
        ph[c].wait()
        compute(rows[c % NB], peb[c % NB], outb[c % NB])
        start_out(c)
        if c + NB < NCHUNK:
            start_gather(c + NB)
            start_pe(c + NB)
    for c in range(NCHUNK - NB, NCHUNK):
        oh[c].wait()


def kernel(tokens, W):
    mesh = plsc.VectorSubcoreMesh(
        core_axis_name="c", subcore_axis_name="s", num_cores=NC, num_subcores=NS
    )
    run = pl.kernel(
        _embed_kernel,
        out_type=jax.ShapeDtypeStruct((B, SEQ, D), jnp.float32),
        mesh=mesh,
        scratch_types=[
            pltpu.VMEM((ROWS_PER_W,), jnp.int32),
            [pltpu.VMEM((C, D), jnp.float32) for _ in range(NB)],
            [pltpu.VMEM((C * D // 2,), jnp.int32) for _ in range(NB)],
            [pltpu.VMEM((C, D), jnp.float32) for _ in range(NB)],
            [pltpu.SemaphoreType.DMA for _ in range(NB)],
            [pltpu.SemaphoreType.DMA for _ in range(NB)],
            [pltpu.SemaphoreType.DMA for _ in range(NB)],
        ],
    )
    return run(tokens.astype(jnp.int32), W, jnp.asarray(_PE))


# speedup vs baseline: 4.4050x; 1.0730x over previous
import math

import ml_dtypes
import numpy as np
import jax
import jax.numpy as jnp
from jax import lax
from jax.experimental import pallas as pl
from jax.experimental.pallas import tpu as pltpu
from jax.experimental.pallas import tpu_sc as plsc

VOCAB = 100000
SEQ = 2048
D = 1024
B = 4
NC = 2
NS = 16
NW = NC * NS
POS_PER_W = SEQ // NW
C = 16
PB = POS_PER_W // C
NCHUNK = PB * B
NB = 3
NB_PE = 2
SCALE = math.sqrt(D)


def _pe_table() -> np.ndarray:
    pos = np.arange(SEQ, dtype=np.float32)[:, None]
    div = np.exp(np.arange(0, D, 2, dtype=np.float32) * (-math.log(10000.0) / D))
    pe = np.zeros((SEQ, D), np.float32)
    pe[:, 0::2] = np.sin(pos * div)
    pe[:, 1::2] = np.cos(pos * div)
    pe = pe.reshape(SEQ * D).astype(ml_dtypes.bfloat16)
    bits = pe.view(np.uint16).reshape(-1, 2, 16).astype(np.uint32)
    words = bits[:, 0, :] | (bits[:, 1, :] << 16)
    return words.reshape(SEQ * D // 2).view(np.int32)


_PE = _pe_table()


def _embed_kernel(tok_hbm, w_hbm, pe_hbm, out_hbm, idx_v, rows, peb, outb, gsems, psems, osems):
    wid = lax.axis_index("s") * NC + lax.axis_index("c")
    pos_base = wid * POS_PER_W

    for s in range(B):
        pltpu.sync_copy(tok_hbm.at[s, pl.ds(pos_base, POS_PER_W)], idx_v.at[s])

    gh, ph, oh = {}, {}, {}

    def chunk_sp(c):
        return c % B, c // B

    def start_gather(c):
        s, pb = chunk_sp(c)
        idx = idx_v.at[s, pl.ds(pb * C, C)]
        gh[c] = pltpu.async_copy(w_hbm.at[idx], rows[c % NB], gsems[c % NB])

    def start_pe(pb):
        src = pe_hbm.at[pl.ds((pos_base + pb * C) * (D // 2), C * D // 2)]
        ph[pb] = pltpu.async_copy(src, peb[pb % NB_PE], psems[pb % NB_PE])

    def start_out(c):
        s, pb = chunk_sp(c)
        dst = out_hbm.at[s, pl.ds(pos_base + pb * C, C)]
        oh[c] = pltpu.async_copy(outb[c % NB], dst, osems[c % NB])

    def compute(rv, pv, ov):
        def rbody(r, carry):
            @plsc.parallel_loop(0, D // 32, unroll=4)
            def jbody(k):
                off = k * 32
                w = pv[pl.ds((r * D + off) // 2, 16)]
                a = lax.bitcast_convert_type(lax.shift_left(w, 16), jnp.float32)
                b = lax.bitcast_convert_type(
                    lax.bitwise_and(w, jnp.int32(-65536)), jnp.float32)
                v0 = rv[r, pl.ds(off, 16)]
                v1 = rv[r, pl.ds(off + 16, 16)]
                ov[r, pl.ds(off, 16)] = a + v0 * SCALE
                ov[r, pl.ds(off + 16, 16)] = b + v1 * SCALE
            return carry
        lax.fori_loop(0, C, rbody, None)

    for c in range(NB):
        start_gather(c)
    start_pe(0)
    start_pe(1)

    for c in range(NCHUNK):
        s, pb = chunk_sp(c)
        if c >= NB:
            oh[c - NB].wait()
        gh[c].wait()
        if s == 0:
            ph[pb].wait()
        compute(rows[c % NB], peb[pb % NB_PE], outb[c % NB])
        start_out(c)
        if c + NB < NCHUNK:
            start_gather(c + NB)
        if s == 0 and pb >= 1 and pb + 1 < PB:
            start_pe(pb + 1)
    for c in range(NCHUNK - NB, NCHUNK):
        oh[c].wait()


def kernel(tokens, W):
    mesh = plsc.VectorSubcoreMesh(
        core_axis_name="c", subcore_axis_name="s", num_cores=NC, num_subcores=NS
    )
    run = pl.kernel(
        _embed_kernel,
        out_type=jax.ShapeDtypeStruct((B, SEQ, D), jnp.float32),
        mesh=mesh,
        scratch_types=[
            pltpu.VMEM((B, POS_PER_W), jnp.int32),
            [pltpu.VMEM((C, D), jnp.float32) for _ in range(NB)],
            [pltpu.VMEM((C * D // 2,), jnp.int32) for _ in range(NB_PE)],
            [pltpu.VMEM((C, D), jnp.float32) for _ in range(NB)],
            [pltpu.SemaphoreType.DMA for _ in range(NB)],
            [pltpu.SemaphoreType.DMA for _ in range(NB_PE)],
            [pltpu.SemaphoreType.DMA for _ in range(NB)],
        ],
    )
    return run(tokens.astype(jnp.int32), W, jnp.asarray(_PE))


# speedup vs baseline: 4.4761x; 1.0161x over previous
import math

import ml_dtypes
import numpy as np
import jax
import jax.numpy as jnp
from jax import lax
from jax.experimental import pallas as pl
from jax.experimental.pallas import tpu as pltpu
from jax.experimental.pallas import tpu_sc as plsc

VOCAB = 100000
SEQ = 2048
D = 1024
B = 4
NC = 2
NS = 16
NW = NC * NS
POS_PER_W = SEQ // NW
C = 16
PB = POS_PER_W // C
NCHUNK = PB * B
NB = 2
SCALE = math.sqrt(D)


def _pe_table() -> np.ndarray:
    pos = np.arange(SEQ, dtype=np.float32)[:, None]
    div = np.exp(np.arange(0, D, 2, dtype=np.float32) * (-math.log(10000.0) / D))
    pe = np.zeros((SEQ, D), np.float32)
    pe[:, 0::2] = np.sin(pos * div)
    pe[:, 1::2] = np.cos(pos * div)
    pe = pe.reshape(SEQ * D).astype(ml_dtypes.bfloat16)
    bits = pe.view(np.uint16).reshape(-1, 2, 16).astype(np.uint32)
    words = bits[:, 0, :] | (bits[:, 1, :] << 16)
    return words.reshape(SEQ * D // 2).view(np.int32)


_PE = _pe_table()


def _embed_kernel(tok_hbm, w_hbm, pe_hbm, out_hbm, idx_v, peb, rows, outb, psem, gsems, osems):
    wid = lax.axis_index("s") * NC + lax.axis_index("c")
    pos_base = wid * POS_PER_W

    pe_desc = pltpu.async_copy(
        pe_hbm.at[pl.ds(pos_base * (D // 2), POS_PER_W * D // 2)], peb, psem)

    for s in range(B):
        pltpu.sync_copy(tok_hbm.at[s, pl.ds(pos_base, POS_PER_W)], idx_v.at[s])

    def sp(c):
        return lax.rem(c, B), lax.div(c, B)

    def gather_desc(c, b):
        s, pb = sp(c)
        idx = idx_v.at[s, pl.ds(pb * C, C)]
        return pltpu.make_async_copy(w_hbm.at[idx], rows[b], gsems[b])

    def out_desc(c, b):
        s, pb = sp(c)
        dst = out_hbm.at[s, pl.ds(pos_base + pb * C, C)]
        return pltpu.make_async_copy(outb[b], dst, osems[b])

    def compute(c, b):
        rv, ov = rows[b], outb[b]
        _, pb = sp(c)
        pe_row0 = pb * C

        def rbody(r, carry):
            base = (pe_row0 + r) * (D // 2)

            @plsc.parallel_loop(0, D // 32, unroll=4)
            def jbody(k):
                off = k * 32
                w = peb[pl.ds(base + k * 16, 16)]
                a = lax.bitcast_convert_type(lax.shift_left(w, 16), jnp.float32)
                b2 = lax.bitcast_convert_type(
                    lax.bitwise_and(w, jnp.int32(-65536)), jnp.float32)
                v0 = rv[r, pl.ds(off, 16)]
                v1 = rv[r, pl.ds(off + 16, 16)]
                ov[r, pl.ds(off, 16)] = a + v0 * SCALE
                ov[r, pl.ds(off + 16, 16)] = b2 + v1 * SCALE
            return carry
        lax.fori_loop(0, C, rbody, None)

    def body(c, b, first, last):
        if not first:
            out_desc(c - NB, b).wait()
        gather_desc(c, b).wait()
        compute(c, b)
        out_desc(c, b).start()
        if not last:
            gather_desc(c + NB, b).start()

    gather_desc(0, 0).start()
    gather_desc(1, 1).start()
    pe_desc.wait()

    body(0, 0, True, False)
    body(1, 1, True, False)

    def group(g, carry):
        c = NB * g
        body(c, 0, False, False)
        body(c + 1, 1, False, False)
        return carry
    lax.fori_loop(1, NCHUNK // NB - 1, group, None)

    body(NCHUNK - 2, 0, False, True)
    body(NCHUNK - 1, 1, False, True)
    out_desc(NCHUNK - 2, 0).wait()
    out_desc(NCHUNK - 1, 1).wait()


def kernel(tokens, W):
    mesh = plsc.VectorSubcoreMesh(
        core_axis_name="c", subcore_axis_name="s", num_cores=NC, num_subcores=NS
    )
    run = pl.kernel(
        _embed_kernel,
        out_type=jax.ShapeDtypeStruct((B, SEQ, D), jnp.float32),
        mesh=mesh,
        scratch_types=[
            pltpu.VMEM((B, POS_PER_W), jnp.int32),
            pltpu.VMEM((POS_PER_W * D // 2,), jnp.int32),
            [pltpu.VMEM((C, D), jnp.float32) for _ in range(NB)],
            [pltpu.VMEM((C, D), jnp.float32) for _ in range(NB)],
            pltpu.SemaphoreType.DMA,
            [pltpu.SemaphoreType.DMA for _ in range(NB)],
            [pltpu.SemaphoreType.DMA for _ in range(NB)],
        ],
    )
    return run(tokens.astype(jnp.int32), W, jnp.asarray(_PE))


# speedup vs baseline: 4.4864x; 1.0023x over previous
import math

import ml_dtypes
import numpy as np
import jax
import jax.numpy as jnp
from jax import lax
from jax.experimental import pallas as pl
from jax.experimental.pallas import tpu as pltpu
from jax.experimental.pallas import tpu_sc as plsc

VOCAB = 100000
SEQ = 2048
D = 1024
B = 4
NC = 2
NS = 16
NW = NC * NS
POS_PER_W = SEQ // NW
C = 16
PB = POS_PER_W // C
NCHUNK = PB * B
NB_R = 3
NB_O = 2
SCALE = math.sqrt(D)


def _pe_table() -> np.ndarray:
    pos = np.arange(SEQ, dtype=np.float32)[:, None]
    div = np.exp(np.arange(0, D, 2, dtype=np.float32) * (-math.log(10000.0) / D))
    pe = np.zeros((SEQ, D), np.float32)
    pe[:, 0::2] = np.sin(pos * div)
    pe[:, 1::2] = np.cos(pos * div)
    pe = pe.reshape(SEQ * D).astype(ml_dtypes.bfloat16)
    bits = pe.view(np.uint16).reshape(-1, 2, 16).astype(np.uint32)
    words = bits[:, 0, :] | (bits[:, 1, :] << 16)
    return words.reshape(SEQ * D // 2).view(np.int32)


_PE = _pe_table()


def _embed_kernel(tok_hbm, w_hbm, pe_hbm, out_hbm, idx_v, peb, rows, outb, psem, gsems, osems):
    wid = lax.axis_index("s") * NC + lax.axis_index("c")
    pos_base = wid * POS_PER_W

    pe_desc = pltpu.async_copy(
        pe_hbm.at[pl.ds(pos_base * (D // 2), POS_PER_W * D // 2)], peb, psem)

    for s in range(B):
        pltpu.sync_copy(tok_hbm.at[s, pl.ds(pos_base, POS_PER_W)], idx_v.at[s])

    gh, oh = {}, {}

    def sp(c):
        return c % B, c // B

    def start_gather(c):
        s, pb = sp(c)
        idx = idx_v.at[s, pl.ds(pb * C, C)]
        gh[c] = pltpu.async_copy(w_hbm.at[idx], rows[c % NB_R], gsems[c % NB_R])

    def start_out(c):
        s, pb = sp(c)
        dst = out_hbm.at[s, pl.ds(pos_base + pb * C, C)]
        oh[c] = pltpu.async_copy(outb[c % NB_O], dst, osems[c % NB_O])

    def compute(c):
        rv, ov = rows[c % NB_R], outb[c % NB_O]
        _, pb = sp(c)
        pe_row0 = pb * C

        def rbody(r, carry):
            base = (pe_row0 + r) * (D // 2)

            @plsc.parallel_loop(0, D // 32, unroll=4)
            def jbody(k):
                off = k * 32
                w = peb[pl.ds(base + k * 16, 16)]
                a = lax.bitcast_convert_type(lax.shift_left(w, 16), jnp.float32)
                b2 = lax.bitcast_convert_type(
                    lax.bitwise_and(w, jnp.int32(-65536)), jnp.float32)
                v0 = rv[r, pl.ds(off, 16)]
                v1 = rv[r, pl.ds(off + 16, 16)]
                ov[r, pl.ds(off, 16)] = a + v0 * SCALE
                ov[r, pl.ds(off + 16, 16)] = b2 + v1 * SCALE
            return carry
        lax.fori_loop(0, C, rbody, None)

    for c in range(NB_R):
        start_gather(c)
    pe_desc.wait()

    for c in range(NCHUNK):
        if c >= NB_O:
            oh[c - NB_O].wait()
        gh[c].wait()
        compute(c)
        start_out(c)
        if c + NB_R < NCHUNK:
            start_gather(c + NB_R)
    for c in range(NCHUNK - NB_O, NCHUNK):
        oh[c].wait()


def kernel(tokens, W):
    mesh = plsc.VectorSubcoreMesh(
        core_axis_name="c", subcore_axis_name="s", num_cores=NC, num_subcores=NS
    )
    run = pl.kernel(
        _embed_kernel,
        out_type=jax.ShapeDtypeStruct((B, SEQ, D), jnp.float32),
        mesh=mesh,
        scratch_types=[
            pltpu.VMEM((B, POS_PER_W), jnp.int32),
            pltpu.VMEM((POS_PER_W * D // 2,), jnp.int32),
            [pltpu.VMEM((C, D), jnp.float32) for _ in range(NB_R)],
            [pltpu.VMEM((C, D), jnp.float32) for _ in range(NB_O)],
            pltpu.SemaphoreType.DMA,
            [pltpu.SemaphoreType.DMA for _ in range(NB_R)],
            [pltpu.SemaphoreType.DMA for _ in range(NB_O)],
        ],
    )
    return run(tokens.astype(jnp.int32), W, jnp.asarray(_PE))
